# baseline (speedup 1.0000x reference)
import functools

import jax
import jax.numpy as jnp
from jax import lax
from jax.experimental import pallas as pl
from jax.experimental.pallas import tpu as pltpu
from jax.experimental.pallas import tpu_sc as plsc

N = 10000
E = 320000
D = 128
NC, NS = 2, 16
NW = NC * NS
NPAD = 10240
LOOK_CHUNK = 80
EPAD = 327680
EDGE_PER_W = EPAD // NW
EW1 = E // NW
ECHUNK = 128
ESTEPS = EDGE_PER_W // ECHUNK
NBUF = 2
TRASH = 10200
HCHUNK = 2000
GPAD = 10240
ROWS_PER_S = GPAD // NS
ZROWS = 64

@functools.cache
def _build_sc_lookup_hist():
    mesh = plsc.VectorSubcoreMesh(core_axis_name="c", subcore_axis_name="s",
                                  num_cores=NC, num_subcores=NS)
    return functools.partial(
        pl.kernel,
        out_type=(
            jax.ShapeDtypeStruct((NPAD, D), jnp.float32),
            jax.ShapeDtypeStruct((NPAD, D), jnp.float32),
            jax.ShapeDtypeStruct((NW, 4, NPAD), jnp.float32),
        ),
        mesh=mesh,
        scratch_types=[
            pltpu.VMEM((LOOK_CHUNK,), jnp.int32),
            pltpu.VMEM((LOOK_CHUNK, D), jnp.float32),
            pltpu.VMEM((HCHUNK,), jnp.int32),
            pltpu.VMEM((NPAD,), jnp.float32),
            pltpu.VMEM((NPAD,), jnp.float32),
            pltpu.VMEM((NPAD,), jnp.float32),
            pltpu.VMEM((NPAD,), jnp.float32),
            pltpu.SemaphoreType.DMA,
        ],
        compiler_params=pltpu.CompilerParams(needs_layout_passes=False),
    )(_sc_lookup_hist_body)


def _sc_lookup_hist_body(usr_table, itm_table, usr_nid, itm_nid,
                    i_src, i_dst, u_src, u_dst,
                    usr_x, itm_x, hist_out,
                    idx_v, rows_v, hidx_v, h0, h1, h2, h3, sem):
    w = lax.axis_index("s") * NC + lax.axis_index("c")

    for tab, nid, out in ((usr_table, usr_nid, usr_x),
                          (itm_table, itm_nid, itm_x)):
        def look_body(j, _, tab=tab, nid=nid, out=out):
            base = w * (NPAD // NW) + j * LOOK_CHUNK
            pltpu.sync_copy(nid.at[pl.ds(base, LOOK_CHUNK)], idx_v)
            pltpu.async_copy(tab.at[idx_v], rows_v, sem).wait()
            pltpu.sync_copy(rows_v, out.at[pl.ds(base, LOOK_CHUNK)])
            return 0
        lax.fori_loop(0, (NPAD // NW) // LOOK_CHUNK, look_body, 0)

    zeros16 = jnp.zeros((16,), jnp.float32)
    ones16 = jnp.ones((16,), jnp.float32)
    hists = (h0, h1, h2, h3)
    for h in hists:
        def zero_body(i, _, h=h):
            h[pl.ds(i * 16, 16)] = zeros16
            return 0
        lax.fori_loop(0, NPAD // 16, zero_body, 0)

    for h, arr in zip(hists, (i_src, i_dst, u_src, u_dst)):
        for k in range(EW1 // HCHUNK):
            pltpu.sync_copy(arr.at[pl.ds(w * EW1 + k * HCHUNK, HCHUNK)],
                            hidx_v)
            def acc_body(i, _, h=h):
                iv = hidx_v[pl.ds(i * 16, 16)]
                plsc.addupdate_scatter(h, [iv], ones16)
                return 0
            lax.fori_loop(0, HCHUNK // 16, acc_body, 0)

    for a, h in enumerate(hists):
        pltpu.sync_copy(h, hist_out.at[w, a])


def _tc_degscale_body(hist_ref, r_ref):
    deg = jnp.sum(hist_ref[...], axis=0)
    r_ref[...] = lax.rsqrt(jnp.maximum(deg, 1.0))


def _tc_degscale(hists):
    return pl.pallas_call(
        _tc_degscale_body,
        out_shape=jax.ShapeDtypeStruct((4, NPAD), jnp.float32),
    )(hists)


def _tc_scale_body(itm_x_ref, usr_x_ref, a_itm_ref, a_usr_ref,
                   h_itm_ref, h_usr_ref):
    h_itm_ref[...] = itm_x_ref[...] * a_itm_ref[...]
    h_usr_ref[...] = usr_x_ref[...] * a_usr_ref[...]


def _tc_scale(itm_x, usr_x, a_itm_col, a_usr_col):
    rows = 2000
    grid = N // rows
    rowspec = pl.BlockSpec((rows, D), lambda i: (i, 0))
    colspec = pl.BlockSpec((rows, 1), lambda i: (i, 0))
    return pl.pallas_call(
        _tc_scale_body,
        grid=(grid,),
        in_specs=[rowspec, rowspec, colspec, colspec],
        out_specs=[rowspec, rowspec],
        out_shape=[jax.ShapeDtypeStruct((N, D), jnp.float32)] * 2,
    )(itm_x, usr_x, a_itm_col, a_usr_col)


@functools.cache
def _build_sc_edge_pass():
    mesh = plsc.VectorSubcoreMesh(core_axis_name="c", subcore_axis_name="s",
                                  num_cores=NC, num_subcores=NS)
    return functools.partial(
        pl.kernel,
        out_type=(
            jax.ShapeDtypeStruct((NC, GPAD, D), jnp.float32),
            jax.ShapeDtypeStruct((NC, GPAD, D), jnp.float32),
        ),
        mesh=mesh,
        scratch_types=[
            pltpu.VMEM_SHARED((GPAD, D), jnp.float32),
            pltpu.VMEM((ZROWS, D), jnp.float32),
        ] + [pltpu.VMEM((ECHUNK, D), jnp.float32)] * NBUF
          + [pltpu.VMEM((ECHUNK,), jnp.int32)] * (2 * NBUF)
          + [pltpu.SemaphoreType.DMA] * (3 * NBUF),
        compiler_params=pltpu.CompilerParams(needs_layout_passes=False),
    )(_sc_edge_pass_body)


def _sc_edge_pass_body(h_itm, h_usr, i_src, i_dst, u_src, u_dst,
                       g0_u, g0_i, acc, zbuf, *bufs_and_sems):
    erows = bufs_and_sems[:NBUF]
    sbuf = bufs_and_sems[NBUF:2 * NBUF]
    dbuf = bufs_and_sems[2 * NBUF:3 * NBUF]
    sems = bufs_and_sems[3 * NBUF:]
    gsem = sems[:NBUF]
    isem = sems[NBUF:2 * NBUF]
    dsem = sems[2 * NBUF:3 * NBUF]
    c = lax.axis_index("c")
    s = lax.axis_index("s")
    w = s * NC + c

    for hsrc, esrc, edst, gout in ((h_itm, i_src, i_dst, g0_u),
                                   (h_usr, u_src, u_dst, g0_i)):
        def zb_body(i, _):
            zbuf[i // (D // 16), pl.ds((i % (D // 16)) * 16, 16)] = (
                jnp.zeros((16,), jnp.float32))
            return 0
        lax.fori_loop(0, ZROWS * (D // 16), zb_body, 0)
        for m in range(ROWS_PER_S // ZROWS):
            pltpu.sync_copy(zbuf, acc.at[pl.ds(s * ROWS_PER_S + m * ZROWS,
                                               ZROWS)])
        plsc.subcore_barrier()

        for k in range(NBUF):
            pltpu.sync_copy(esrc.at[pl.ds(w * EDGE_PER_W + k * ECHUNK,
                                          ECHUNK)], sbuf[k])
            pltpu.sync_copy(edst.at[pl.ds(w * EDGE_PER_W + k * ECHUNK,
                                          ECHUNK)], dbuf[k])
            pltpu.async_copy(hsrc.at[sbuf[k]], erows[k], gsem[k])

        def turn(j, k, reload, hsrc, esrc, edst):
            pltpu.make_async_copy(hsrc.at[sbuf[k]], erows[k], gsem[k]).wait()
            if reload:
                nbase = w * EDGE_PER_W + (j + NBUF) * ECHUNK
                pltpu.async_copy(esrc.at[pl.ds(nbase, ECHUNK)], sbuf[k],
                                 isem[k])
            pltpu.sync_copy(erows[k], acc.at[dbuf[k]], add=True)
            if reload:
                nbase = w * EDGE_PER_W + (j + NBUF) * ECHUNK
                pltpu.async_copy(edst.at[pl.ds(nbase, ECHUNK)], dbuf[k],
                                 dsem[k])
                pltpu.make_async_copy(esrc.at[pl.ds(nbase, ECHUNK)], sbuf[k],
                                      isem[k]).wait()
                pltpu.async_copy(hsrc.at[sbuf[k]], erows[k], gsem[k])
                pltpu.make_async_copy(edst.at[pl.ds(nbase, ECHUNK)], dbuf[k],
                                      dsem[k]).wait()

        def outer(o, _, hsrc=hsrc, esrc=esrc, edst=edst):
            for k in range(NBUF):
                turn(o * NBUF + k, k, True, hsrc, esrc, edst)
            return 0
        n_main = ESTEPS // NBUF - 1
        lax.fori_loop(0, n_main, outer, 0)
        for k in range(NBUF):
            turn(n_main * NBUF + k, k, False, hsrc, esrc, edst)
        plsc.subcore_barrier()

        for m in range(ROWS_PER_S // ZROWS):
            row0 = s * ROWS_PER_S + m * ZROWS
            pltpu.sync_copy(acc.at[pl.ds(row0, ZROWS)], zbuf)
            pltpu.sync_copy(zbuf, gout.at[c, pl.ds(row0, ZROWS)])
        plsc.subcore_barrier()


def _tc_final_body(g0u_ref, g0i_ref, usr_x_ref, itm_x_ref, bu_ref, bi_ref,
                   wi_t_ref, wt_t_ref, new_usr_ref, new_itm_ref):
    wi_t = wi_t_ref[...]
    wt_t = wt_t_ref[...]
    for g0_ref, x_ref, b_ref, out_ref in (
            (g0u_ref, usr_x_ref, bu_ref, new_usr_ref),
            (g0i_ref, itm_x_ref, bi_ref, new_itm_ref)):
        x = x_ref[...]
        g = (g0_ref[0] + g0_ref[1]) * b_ref[...]
        z = (jnp.dot(x + g, wi_t, preferred_element_type=jnp.float32)
             + jnp.dot(g * x, wt_t, preferred_element_type=jnp.float32))
        out_ref[...] = 0.5 * x + 0.5 * jnp.where(z >= 0, z, 0.01 * z)


def _tc_final(g0u, g0i, usr_x, itm_x, bu_col, bi_col, wi_t, wt_t):
    rows = 2000
    grid = N // rows
    gspec = pl.BlockSpec((NC, rows, D), lambda i: (0, i, 0))
    rowspec = pl.BlockSpec((rows, D), lambda i: (i, 0))
    colspec = pl.BlockSpec((rows, 1), lambda i: (i, 0))
    wspec = pl.BlockSpec((D, D), lambda i: (0, 0))
    return pl.pallas_call(
        _tc_final_body,
        grid=(grid,),
        in_specs=[gspec, gspec, rowspec, rowspec, colspec, colspec,
                  wspec, wspec],
        out_specs=[rowspec, rowspec],
        out_shape=[jax.ShapeDtypeStruct((N, D), jnp.float32)] * 2,
    )(g0u, g0i, usr_x, itm_x, bu_col, bi_col, wi_t, wt_t)


def kernel(usr_table, itm_table, W_intra_user, W_inter_user,
           W_intra_item, W_inter_item,
           usr_n_id, itm_n_id, usr_edge_index, itm_edge_index):
    del W_intra_item, W_inter_item
    usr_n_id = usr_n_id.astype(jnp.int32)
    itm_n_id = itm_n_id.astype(jnp.int32)

    pad = NPAD - N
    usr_nid_p = jnp.pad(usr_n_id, (0, pad))
    itm_nid_p = jnp.pad(itm_n_id, (0, pad))
    i_src = itm_edge_index[0].astype(jnp.int32)
    i_dst = itm_edge_index[1].astype(jnp.int32)
    u_src = usr_edge_index[0].astype(jnp.int32)
    u_dst = usr_edge_index[1].astype(jnp.int32)
    epad = EPAD - E
    i_src_p = jnp.pad(i_src, (0, epad))
    i_dst_p = jnp.pad(i_dst, (0, epad), constant_values=TRASH)
    u_src_p = jnp.pad(u_src, (0, epad))
    u_dst_p = jnp.pad(u_dst, (0, epad), constant_values=TRASH)

    usr_x_p, itm_x_p, hists = _build_sc_lookup_hist()(
        usr_table, itm_table, usr_nid_p, itm_nid_p,
        i_src, i_dst, u_src, u_dst)
    usr_x = usr_x_p[:N]
    itm_x = itm_x_p[:N]

    r = _tc_degscale(hists)
    a_itm_col = r[0, :N].reshape(N, 1)
    b_u_col = r[1, :N].reshape(N, 1)
    a_usr_col = r[2, :N].reshape(N, 1)
    b_i_col = r[3, :N].reshape(N, 1)

    h_itm, h_usr = _tc_scale(itm_x, usr_x, a_itm_col, a_usr_col)

    g0_u_p, g0_i_p = _build_sc_edge_pass()(h_itm, h_usr, i_src_p, i_dst_p,
                                           u_src_p, u_dst_p)
    g0_u = g0_u_p[:, :N]
    g0_i = g0_i_p[:, :N]

    new_usr, new_itm = _tc_final(
        g0_u, g0_i, usr_x, itm_x, b_u_col, b_i_col,
        W_intra_user.T, W_inter_user.T)
    return (new_usr, new_itm)


# speedup vs baseline: 2.4877x; 2.4877x over previous
import functools

import jax
import jax.numpy as jnp
from jax import lax
from jax.experimental import pallas as pl
from jax.experimental.pallas import tpu as pltpu
from jax.experimental.pallas import tpu_sc as plsc

N = 10000
E = 320000
D = 128
NC, NS = 2, 16
NW = NC * NS
NPAD = 10240
LOOK_CHUNK = 80
EPAD = 327680
EDGE_PER_W = EPAD // NW
EW1 = E // NW
ECHUNK = 128
ESTEPS = EDGE_PER_W // ECHUNK
NBUF = 2
HCHUNK = 2000
GPAD = 10240
ROWS_PER_S = GPAD // NS
ZROWS = 64

@functools.cache
def _build_sc_lookup_hist():
    mesh = plsc.VectorSubcoreMesh(core_axis_name="c", subcore_axis_name="s",
                                  num_cores=NC, num_subcores=NS)
    return functools.partial(
        pl.kernel,
        out_type=(
            jax.ShapeDtypeStruct((NPAD, D), jnp.float32),
            jax.ShapeDtypeStruct((NPAD, D), jnp.float32),
            jax.ShapeDtypeStruct((NW, 4, NPAD), jnp.float32),
        ),
        mesh=mesh,
        scratch_types=[
            pltpu.VMEM((LOOK_CHUNK,), jnp.int32),
            pltpu.VMEM((LOOK_CHUNK, D), jnp.float32),
            pltpu.VMEM((HCHUNK,), jnp.int32),
            pltpu.VMEM((NPAD,), jnp.float32),
            pltpu.VMEM((NPAD,), jnp.float32),
            pltpu.VMEM((NPAD,), jnp.float32),
            pltpu.VMEM((NPAD,), jnp.float32),
            pltpu.SemaphoreType.DMA,
        ],
        compiler_params=pltpu.CompilerParams(needs_layout_passes=False),
    )(_sc_lookup_hist_body)


def _sc_lookup_hist_body(usr_table, itm_table, usr_nid, itm_nid,
                    i_src, i_dst, u_src, u_dst,
                    usr_x, itm_x, hist_out,
                    idx_v, rows_v, hidx_v, h0, h1, h2, h3, sem):
    w = lax.axis_index("s") * NC + lax.axis_index("c")

    for tab, nid, out in ((usr_table, usr_nid, usr_x),
                          (itm_table, itm_nid, itm_x)):
        def look_body(j, _, tab=tab, nid=nid, out=out):
            base = w * (NPAD // NW) + j * LOOK_CHUNK
            pltpu.sync_copy(nid.at[pl.ds(base, LOOK_CHUNK)], idx_v)
            pltpu.async_copy(tab.at[idx_v], rows_v, sem).wait()
            pltpu.sync_copy(rows_v, out.at[pl.ds(base, LOOK_CHUNK)])
            return 0
        lax.fori_loop(0, (NPAD // NW) // LOOK_CHUNK, look_body, 0)

    zeros16 = jnp.zeros((16,), jnp.float32)
    ones16 = jnp.ones((16,), jnp.float32)
    hists = (h0, h1, h2, h3)
    for h in hists:
        def zero_body(i, _, h=h):
            h[pl.ds(i * 16, 16)] = zeros16
            return 0
        lax.fori_loop(0, NPAD // 16, zero_body, 0)

    for h, arr in zip(hists, (i_src, i_dst, u_src, u_dst)):
        for k in range(EW1 // HCHUNK):
            pltpu.sync_copy(arr.at[pl.ds(w * EW1 + k * HCHUNK, HCHUNK)],
                            hidx_v)
            def acc_body(i, _, h=h):
                iv = hidx_v[pl.ds(i * 16, 16)]
                plsc.addupdate_scatter(h, [iv], ones16)
                return 0
            lax.fori_loop(0, HCHUNK // 16, acc_body, 0)

    for a, h in enumerate(hists):
        pltpu.sync_copy(h, hist_out.at[w, a])


def _tc_degscale_body(hist_ref, r_ref):
    deg = jnp.sum(hist_ref[...], axis=0)
    r_ref[...] = lax.rsqrt(jnp.maximum(deg, 1.0))


def _tc_degscale(hists):
    return pl.pallas_call(
        _tc_degscale_body,
        out_shape=jax.ShapeDtypeStruct((4, NPAD), jnp.float32),
    )(hists)


def _tc_scale_body(itm_x_ref, usr_x_ref, a_itm_ref, a_usr_ref,
                   h_itm_ref, h_usr_ref):
    h_itm_ref[...] = itm_x_ref[...] * a_itm_ref[...]
    h_usr_ref[...] = usr_x_ref[...] * a_usr_ref[...]


def _tc_scale(itm_x, usr_x, a_itm_col, a_usr_col):
    rows = 2000
    grid = N // rows
    rowspec = pl.BlockSpec((rows, D), lambda i: (i, 0))
    colspec = pl.BlockSpec((rows, 1), lambda i: (i, 0))
    return pl.pallas_call(
        _tc_scale_body,
        grid=(grid,),
        in_specs=[rowspec, rowspec, colspec, colspec],
        out_specs=[rowspec, rowspec],
        out_shape=[jax.ShapeDtypeStruct((N, D), jnp.float32)] * 2,
    )(itm_x, usr_x, a_itm_col, a_usr_col)


@functools.cache
def _build_sc_edge_pass():
    mesh = plsc.VectorSubcoreMesh(core_axis_name="c", subcore_axis_name="s",
                                  num_cores=NC, num_subcores=NS)
    return functools.partial(
        pl.kernel,
        out_type=(
            jax.ShapeDtypeStruct((NC, GPAD, D), jnp.float32),
            jax.ShapeDtypeStruct((NC, GPAD, D), jnp.float32),
        ),
        mesh=mesh,
        scratch_types=[
            pltpu.VMEM_SHARED((GPAD, D), jnp.float32),
            pltpu.VMEM((ZROWS, D), jnp.float32),
        ] + [pltpu.VMEM((ECHUNK, D), jnp.float32)] * NBUF
          + [pltpu.VMEM((ECHUNK,), jnp.int32)] * (2 * NBUF)
          + [pltpu.SemaphoreType.DMA] * (3 * NBUF),
        compiler_params=pltpu.CompilerParams(needs_layout_passes=False),
    )(_sc_edge_pass_body)


def _sc_edge_pass_body(h_itm, h_usr, i_src, i_dst, u_src, u_dst,
                       g0_u, g0_i, acc, zbuf, *bufs_and_sems):
    erows = bufs_and_sems[:NBUF]
    sbuf = bufs_and_sems[NBUF:2 * NBUF]
    dbuf = bufs_and_sems[2 * NBUF:3 * NBUF]
    sems = bufs_and_sems[3 * NBUF:]
    gsem = sems[:NBUF]
    isem = sems[NBUF:2 * NBUF]
    dsem = sems[2 * NBUF:3 * NBUF]
    c = lax.axis_index("c")
    s = lax.axis_index("s")
    w = s * NC + c

    for hsrc, esrc, edst, gout in ((h_itm, i_src, i_dst, g0_u),
                                   (h_usr, u_src, u_dst, g0_i)):
        def zb_body(i, _):
            zbuf[i // (D // 16), pl.ds((i % (D // 16)) * 16, 16)] = (
                jnp.zeros((16,), jnp.float32))
            return 0
        lax.fori_loop(0, ZROWS * (D // 16), zb_body, 0)
        for m in range(ROWS_PER_S // ZROWS):
            pltpu.sync_copy(zbuf, acc.at[pl.ds(s * ROWS_PER_S + m * ZROWS,
                                               ZROWS)])
        plsc.subcore_barrier()

        for k in range(NBUF):
            pltpu.sync_copy(esrc.at[pl.ds(w * EDGE_PER_W + k * ECHUNK,
                                          ECHUNK)], sbuf[k])
            pltpu.sync_copy(edst.at[pl.ds(w * EDGE_PER_W + k * ECHUNK,
                                          ECHUNK)], dbuf[k])
            pltpu.async_copy(hsrc.at[sbuf[k]], erows[k], gsem[k])

        def turn(j, k, reload, hsrc, esrc, edst):
            pltpu.make_async_copy(hsrc.at[sbuf[k]], erows[k], gsem[k]).wait()
            if reload:
                nbase = w * EDGE_PER_W + (j + NBUF) * ECHUNK
                pltpu.async_copy(esrc.at[pl.ds(nbase, ECHUNK)], sbuf[k],
                                 isem[k])
            pltpu.sync_copy(erows[k], acc.at[dbuf[k]], add=True)
            if reload:
                nbase = w * EDGE_PER_W + (j + NBUF) * ECHUNK
                pltpu.async_copy(edst.at[pl.ds(nbase, ECHUNK)], dbuf[k],
                                 dsem[k])
                pltpu.make_async_copy(esrc.at[pl.ds(nbase, ECHUNK)], sbuf[k],
                                      isem[k]).wait()
                pltpu.async_copy(hsrc.at[sbuf[k]], erows[k], gsem[k])
                pltpu.make_async_copy(edst.at[pl.ds(nbase, ECHUNK)], dbuf[k],
                                      dsem[k]).wait()

        def outer(o, _, hsrc=hsrc, esrc=esrc, edst=edst):
            for k in range(NBUF):
                turn(o * NBUF + k, k, True, hsrc, esrc, edst)
            return 0
        n_main = ESTEPS // NBUF - 1
        lax.fori_loop(0, n_main, outer, 0)
        for k in range(NBUF):
            turn(n_main * NBUF + k, k, False, hsrc, esrc, edst)
        plsc.subcore_barrier()

        for m in range(ROWS_PER_S // ZROWS):
            row0 = s * ROWS_PER_S + m * ZROWS
            pltpu.sync_copy(acc.at[pl.ds(row0, ZROWS)], zbuf)
            pltpu.sync_copy(zbuf, gout.at[c, pl.ds(row0, ZROWS)])
        plsc.subcore_barrier()


def _tc_final_body(g0u_ref, g0i_ref, usr_x_ref, itm_x_ref, bu_ref, bi_ref,
                   wi_t_ref, wt_t_ref, new_usr_ref, new_itm_ref):
    wi_t = wi_t_ref[...]
    wt_t = wt_t_ref[...]
    for g0_ref, x_ref, b_ref, out_ref in (
            (g0u_ref, usr_x_ref, bu_ref, new_usr_ref),
            (g0i_ref, itm_x_ref, bi_ref, new_itm_ref)):
        x = x_ref[...]
        g = (g0_ref[0] + g0_ref[1]) * b_ref[...]
        z = (jnp.dot(x + g, wi_t, preferred_element_type=jnp.float32)
             + jnp.dot(g * x, wt_t, preferred_element_type=jnp.float32))
        out_ref[...] = 0.5 * x + 0.5 * jnp.where(z >= 0, z, 0.01 * z)


def _tc_final(g0u, g0i, usr_x, itm_x, bu_col, bi_col, wi_t, wt_t):
    rows = 2000
    grid = N // rows
    gspec = pl.BlockSpec((NC, rows, D), lambda i: (0, i, 0))
    rowspec = pl.BlockSpec((rows, D), lambda i: (i, 0))
    colspec = pl.BlockSpec((rows, 1), lambda i: (i, 0))
    wspec = pl.BlockSpec((D, D), lambda i: (0, 0))
    return pl.pallas_call(
        _tc_final_body,
        grid=(grid,),
        in_specs=[gspec, gspec, rowspec, rowspec, colspec, colspec,
                  wspec, wspec],
        out_specs=[rowspec, rowspec],
        out_shape=[jax.ShapeDtypeStruct((N, D), jnp.float32)] * 2,
    )(g0u, g0i, usr_x, itm_x, bu_col, bi_col, wi_t, wt_t)


def kernel(usr_table, itm_table, W_intra_user, W_inter_user,
           W_intra_item, W_inter_item,
           usr_n_id, itm_n_id, usr_edge_index, itm_edge_index):
    del W_intra_item, W_inter_item
    usr_n_id = usr_n_id.astype(jnp.int32)
    itm_n_id = itm_n_id.astype(jnp.int32)

    pad = NPAD - N
    usr_nid_p = jnp.pad(usr_n_id, (0, pad))
    itm_nid_p = jnp.pad(itm_n_id, (0, pad))
    i_src = itm_edge_index[0].astype(jnp.int32)
    i_dst = itm_edge_index[1].astype(jnp.int32)
    u_src = usr_edge_index[0].astype(jnp.int32)
    u_dst = usr_edge_index[1].astype(jnp.int32)
    epad = EPAD - E
    pad_src = (jnp.arange(epad, dtype=jnp.int32) * 37) % N
    pad_dst = N + (jnp.arange(epad, dtype=jnp.int32) % (GPAD - N))
    i_src_p = jnp.concatenate([i_src, pad_src])
    i_dst_p = jnp.concatenate([i_dst, pad_dst])
    u_src_p = jnp.concatenate([u_src, pad_src])
    u_dst_p = jnp.concatenate([u_dst, pad_dst])

    usr_x_p, itm_x_p, hists = _build_sc_lookup_hist()(
        usr_table, itm_table, usr_nid_p, itm_nid_p,
        i_src, i_dst, u_src, u_dst)
    usr_x = usr_x_p[:N]
    itm_x = itm_x_p[:N]

    r = _tc_degscale(hists)
    a_itm_col = r[0, :N].reshape(N, 1)
    b_u_col = r[1, :N].reshape(N, 1)
    a_usr_col = r[2, :N].reshape(N, 1)
    b_i_col = r[3, :N].reshape(N, 1)

    h_itm, h_usr = _tc_scale(itm_x, usr_x, a_itm_col, a_usr_col)

    g0_u_p, g0_i_p = _build_sc_edge_pass()(h_itm, h_usr, i_src_p, i_dst_p,
                                           u_src_p, u_dst_p)
    g0_u = g0_u_p[:, :N]
    g0_i = g0_i_p[:, :N]

    new_usr, new_itm = _tc_final(
        g0_u, g0_i, usr_x, itm_x, b_u_col, b_i_col,
        W_intra_user.T, W_inter_user.T)
    return (new_usr, new_itm)


# speedup vs baseline: 2.7402x; 1.1015x over previous
import functools

import jax
import jax.numpy as jnp
from jax import lax
from jax.experimental import pallas as pl
from jax.experimental.pallas import tpu as pltpu
from jax.experimental.pallas import tpu_sc as plsc

N = 10000
E = 320000
D = 128
NC, NS = 2, 16
NW = NC * NS
NPAD = 10240
LOOK_CHUNK = 80
EPAD = 327680
EDGE_PER_W = EPAD // NW
EW1 = E // NW
ECHUNK = 128
ESTEPS = EDGE_PER_W // ECHUNK
NBUF = 2
HCHUNK = 2000
GPAD = 10240
ROWS_PER_S = GPAD // NS
ZROWS = 64

@functools.cache
def _build_sc_lookup_hist():
    mesh = plsc.VectorSubcoreMesh(core_axis_name="c", subcore_axis_name="s",
                                  num_cores=NC, num_subcores=NS)
    return functools.partial(
        pl.kernel,
        out_type=(
            jax.ShapeDtypeStruct((NPAD, D), jnp.float32),
            jax.ShapeDtypeStruct((NPAD, D), jnp.float32),
            jax.ShapeDtypeStruct((NW, 4, NPAD), jnp.float32),
        ),
        mesh=mesh,
        scratch_types=[
            pltpu.VMEM((NPAD // NW,), jnp.int32),
            pltpu.VMEM((NPAD // NW,), jnp.int32),
            pltpu.VMEM((LOOK_CHUNK, D), jnp.float32),
            pltpu.VMEM((LOOK_CHUNK, D), jnp.float32),
            pltpu.VMEM((EW1,), jnp.int32),
            pltpu.VMEM((EW1,), jnp.int32),
            pltpu.VMEM((NPAD,), jnp.float32),
            pltpu.VMEM((NPAD,), jnp.float32),
            pltpu.VMEM((NPAD,), jnp.float32),
            pltpu.VMEM((NPAD,), jnp.float32),
        ] + [pltpu.SemaphoreType.DMA] * 6,
        compiler_params=pltpu.CompilerParams(needs_layout_passes=False),
    )(_sc_lookup_hist_body)


def _sc_lookup_hist_body(usr_table, itm_table, usr_nid, itm_nid,
                         i_src, i_dst, u_src, u_dst,
                         usr_x, itm_x, hist_out,
                         unid, inid, rows0, rows1, hidx0, hidx1,
                         h0, h1, h2, h3,
                         gsem0, gsem1, osem0, osem1, hsem0, hsem1):
    w = lax.axis_index("s") * NC + lax.axis_index("c")
    npw = NPAD // NW
    rows = (rows0, rows1)
    gsem = (gsem0, gsem1)
    osem = (osem0, osem1)
    hidx = (hidx0, hidx1)
    hsem = (hsem0, hsem1)
    harr = (i_src, i_dst, u_src, u_dst)
    hists = (h0, h1, h2, h3)

    pltpu.async_copy(i_src.at[pl.ds(w * EW1, EW1)], hidx0, hsem0)
    pltpu.sync_copy(usr_nid.at[pl.ds(w * npw, npw)], unid)
    pltpu.sync_copy(itm_nid.at[pl.ds(w * npw, npw)], inid)

    jobs = [(usr_table, unid, usr_x, ci) for ci in range(npw // LOOK_CHUNK)]
    jobs += [(itm_table, inid, itm_x, ci) for ci in range(npw // LOOK_CHUNK)]

    def g_src(p):
        tab, nidv, _, ci = jobs[p]
        return tab.at[nidv.at[pl.ds(ci * LOOK_CHUNK, LOOK_CHUNK)]]

    def o_dst(p):
        _, _, out, ci = jobs[p]
        return out.at[pl.ds(w * npw + ci * LOOK_CHUNK, LOOK_CHUNK)]

    for k in range(2):
        pltpu.async_copy(g_src(k), rows[k], gsem[k])
    for p in range(len(jobs)):
        k = p % 2
        pltpu.make_async_copy(g_src(p), rows[k], gsem[k]).wait()
        pltpu.async_copy(rows[k], o_dst(p), osem[k])
        if p + 2 < len(jobs):
            pltpu.make_async_copy(rows[k], o_dst(p), osem[k]).wait()
            pltpu.async_copy(g_src(p + 2), rows[k], gsem[k])
    for p in (len(jobs) - 2, len(jobs) - 1):
        k = p % 2
        pltpu.make_async_copy(rows[k], o_dst(p), osem[k]).wait()

    zeros16 = jnp.zeros((16,), jnp.float32)
    ones16 = jnp.ones((16,), jnp.float32)
    for h in hists:
        def zero_body(i, _, h=h):
            h[pl.ds(i * 16, 16)] = zeros16
            return 0
        lax.fori_loop(0, NPAD // 16, zero_body, 0)

    for a in range(4):
        k = a % 2
        if a + 1 < 4:
            pltpu.async_copy(harr[a + 1].at[pl.ds(w * EW1, EW1)],
                             hidx[1 - k], hsem[1 - k])
        pltpu.make_async_copy(harr[a].at[pl.ds(w * EW1, EW1)], hidx[k],
                              hsem[k]).wait()

        def acc_body(i, _, h=hists[a], buf=hidx[k]):
            plsc.addupdate_scatter(h, [buf[pl.ds(i * 16, 16)]], ones16)
            return 0
        lax.fori_loop(0, EW1 // 16, acc_body, 0)

    for a, h in enumerate(hists):
        pltpu.async_copy(h, hist_out.at[w, a], hsem0)
    for a, h in enumerate(hists):
        pltpu.make_async_copy(h, hist_out.at[w, a], hsem0).wait()


def _tc_degscale_body(hist_ref, r_ref):
    deg = jnp.sum(hist_ref[...], axis=0)
    r_ref[...] = lax.rsqrt(jnp.maximum(deg, 1.0))


def _tc_degscale(hists):
    return pl.pallas_call(
        _tc_degscale_body,
        out_shape=jax.ShapeDtypeStruct((4, NPAD), jnp.float32),
    )(hists)


def _tc_scale_body(itm_x_ref, usr_x_ref, a_itm_ref, a_usr_ref,
                   h_itm_ref, h_usr_ref):
    h_itm_ref[...] = itm_x_ref[...] * a_itm_ref[...]
    h_usr_ref[...] = usr_x_ref[...] * a_usr_ref[...]


def _tc_scale(itm_x_p, usr_x_p, a_itm_col, a_usr_col):
    rows = 2000
    grid = N // rows
    rowspec = pl.BlockSpec((rows, D), lambda i: (i, 0))
    colspec = pl.BlockSpec((rows, 1), lambda i: (i, 0))
    return pl.pallas_call(
        _tc_scale_body,
        grid=(grid,),
        in_specs=[rowspec, rowspec, colspec, colspec],
        out_specs=[rowspec, rowspec],
        out_shape=[jax.ShapeDtypeStruct((N, D), jnp.float32)] * 2,
    )(itm_x_p, usr_x_p, a_itm_col, a_usr_col)


@functools.cache
def _build_sc_edge_pass():
    mesh = plsc.VectorSubcoreMesh(core_axis_name="c", subcore_axis_name="s",
                                  num_cores=NC, num_subcores=NS)
    return functools.partial(
        pl.kernel,
        out_type=(
            jax.ShapeDtypeStruct((NC, GPAD, D), jnp.float32),
            jax.ShapeDtypeStruct((NC, GPAD, D), jnp.float32),
        ),
        mesh=mesh,
        scratch_types=[
            pltpu.VMEM_SHARED((GPAD, D), jnp.float32),
            pltpu.VMEM((ZROWS, D), jnp.float32),
        ] + [pltpu.VMEM((ECHUNK, D), jnp.float32)] * NBUF
          + [pltpu.VMEM((ECHUNK,), jnp.int32)] * (2 * NBUF)
          + [pltpu.SemaphoreType.DMA] * (3 * NBUF),
        compiler_params=pltpu.CompilerParams(needs_layout_passes=False),
    )(_sc_edge_pass_body)


def _sc_edge_pass_body(h_itm, h_usr, i_src, i_dst, u_src, u_dst,
                       g0_u, g0_i, acc, zbuf, *bufs_and_sems):
    erows = bufs_and_sems[:NBUF]
    sbuf = bufs_and_sems[NBUF:2 * NBUF]
    dbuf = bufs_and_sems[2 * NBUF:3 * NBUF]
    sems = bufs_and_sems[3 * NBUF:]
    gsem = sems[:NBUF]
    isem = sems[NBUF:2 * NBUF]
    dsem = sems[2 * NBUF:3 * NBUF]
    c = lax.axis_index("c")
    s = lax.axis_index("s")
    w = s * NC + c

    for hsrc, esrc, edst, gout in ((h_itm, i_src, i_dst, g0_u),
                                   (h_usr, u_src, u_dst, g0_i)):
        def zb_body(i, _):
            zbuf[i // (D // 16), pl.ds((i % (D // 16)) * 16, 16)] = (
                jnp.zeros((16,), jnp.float32))
            return 0
        lax.fori_loop(0, ZROWS * (D // 16), zb_body, 0)
        for m in range(ROWS_PER_S // ZROWS):
            pltpu.sync_copy(zbuf, acc.at[pl.ds(s * ROWS_PER_S + m * ZROWS,
                                               ZROWS)])
        plsc.subcore_barrier()

        for k in range(NBUF):
            pltpu.sync_copy(esrc.at[pl.ds(w * EDGE_PER_W + k * ECHUNK,
                                          ECHUNK)], sbuf[k])
            pltpu.sync_copy(edst.at[pl.ds(w * EDGE_PER_W + k * ECHUNK,
                                          ECHUNK)], dbuf[k])
            pltpu.async_copy(hsrc.at[sbuf[k]], erows[k], gsem[k])

        def turn(j, k, reload, hsrc, esrc, edst):
            pltpu.make_async_copy(hsrc.at[sbuf[k]], erows[k], gsem[k]).wait()
            if reload:
                nbase = w * EDGE_PER_W + (j + NBUF) * ECHUNK
                pltpu.async_copy(esrc.at[pl.ds(nbase, ECHUNK)], sbuf[k],
                                 isem[k])
            pltpu.sync_copy(erows[k], acc.at[dbuf[k]], add=True)
            if reload:
                nbase = w * EDGE_PER_W + (j + NBUF) * ECHUNK
                pltpu.async_copy(edst.at[pl.ds(nbase, ECHUNK)], dbuf[k],
                                 dsem[k])
                pltpu.make_async_copy(esrc.at[pl.ds(nbase, ECHUNK)], sbuf[k],
                                      isem[k]).wait()
                pltpu.async_copy(hsrc.at[sbuf[k]], erows[k], gsem[k])
                pltpu.make_async_copy(edst.at[pl.ds(nbase, ECHUNK)], dbuf[k],
                                      dsem[k]).wait()

        def outer(o, _, hsrc=hsrc, esrc=esrc, edst=edst):
            for k in range(NBUF):
                turn(o * NBUF + k, k, True, hsrc, esrc, edst)
            return 0
        n_main = ESTEPS // NBUF - 1
        lax.fori_loop(0, n_main, outer, 0)
        for k in range(NBUF):
            turn(n_main * NBUF + k, k, False, hsrc, esrc, edst)
        plsc.subcore_barrier()

        for m in range(ROWS_PER_S // ZROWS):
            row0 = s * ROWS_PER_S + m * ZROWS
            pltpu.sync_copy(acc.at[pl.ds(row0, ZROWS)], zbuf)
            pltpu.sync_copy(zbuf, gout.at[c, pl.ds(row0, ZROWS)])
        plsc.subcore_barrier()


def _tc_final_body(g0u_ref, g0i_ref, usr_x_ref, itm_x_ref, bu_ref, bi_ref,
                   wi_t_ref, wt_t_ref, new_usr_ref, new_itm_ref):
    wi_t = wi_t_ref[...]
    wt_t = wt_t_ref[...]
    for g0_ref, x_ref, b_ref, out_ref in (
            (g0u_ref, usr_x_ref, bu_ref, new_usr_ref),
            (g0i_ref, itm_x_ref, bi_ref, new_itm_ref)):
        x = x_ref[...]
        g = (g0_ref[0] + g0_ref[1]) * b_ref[...]
        z = (jnp.dot(x + g, wi_t, preferred_element_type=jnp.float32)
             + jnp.dot(g * x, wt_t, preferred_element_type=jnp.float32))
        out_ref[...] = 0.5 * x + 0.5 * jnp.where(z >= 0, z, 0.01 * z)


def _tc_final(g0u, g0i, usr_x, itm_x, bu_col, bi_col, wi_t, wt_t):
    rows = 2000
    grid = N // rows
    gspec = pl.BlockSpec((NC, rows, D), lambda i: (0, i, 0))
    rowspec = pl.BlockSpec((rows, D), lambda i: (i, 0))
    colspec = pl.BlockSpec((rows, 1), lambda i: (i, 0))
    wspec = pl.BlockSpec((D, D), lambda i: (0, 0))
    return pl.pallas_call(
        _tc_final_body,
        grid=(grid,),
        in_specs=[gspec, gspec, rowspec, rowspec, colspec, colspec,
                  wspec, wspec],
        out_specs=[rowspec, rowspec],
        out_shape=[jax.ShapeDtypeStruct((N, D), jnp.float32)] * 2,
    )(g0u, g0i, usr_x, itm_x, bu_col, bi_col, wi_t, wt_t)


def kernel(usr_table, itm_table, W_intra_user, W_inter_user,
           W_intra_item, W_inter_item,
           usr_n_id, itm_n_id, usr_edge_index, itm_edge_index):
    del W_intra_item, W_inter_item
    usr_n_id = usr_n_id.astype(jnp.int32)
    itm_n_id = itm_n_id.astype(jnp.int32)

    pad = NPAD - N
    usr_nid_p = jnp.pad(usr_n_id, (0, pad))
    itm_nid_p = jnp.pad(itm_n_id, (0, pad))
    i_src = itm_edge_index[0].astype(jnp.int32)
    i_dst = itm_edge_index[1].astype(jnp.int32)
    u_src = usr_edge_index[0].astype(jnp.int32)
    u_dst = usr_edge_index[1].astype(jnp.int32)
    epad = EPAD - E
    pad_src = (jnp.arange(epad, dtype=jnp.int32) * 37) % N
    pad_dst = N + (jnp.arange(epad, dtype=jnp.int32) % (GPAD - N))
    i_src_p = jnp.concatenate([i_src, pad_src])
    i_dst_p = jnp.concatenate([i_dst, pad_dst])
    u_src_p = jnp.concatenate([u_src, pad_src])
    u_dst_p = jnp.concatenate([u_dst, pad_dst])

    usr_x_p, itm_x_p, hists = _build_sc_lookup_hist()(
        usr_table, itm_table, usr_nid_p, itm_nid_p,
        i_src, i_dst, u_src, u_dst)

    r = _tc_degscale(hists)
    a_itm_col = r[0, :N].reshape(N, 1)
    b_u_col = r[1, :N].reshape(N, 1)
    a_usr_col = r[2, :N].reshape(N, 1)
    b_i_col = r[3, :N].reshape(N, 1)

    h_itm, h_usr = _tc_scale(itm_x_p, usr_x_p, a_itm_col, a_usr_col)

    g0_u_p, g0_i_p = _build_sc_edge_pass()(h_itm, h_usr, i_src_p, i_dst_p,
                                           u_src_p, u_dst_p)

    new_usr, new_itm = _tc_final(
        g0_u_p, g0_i_p, usr_x_p, itm_x_p, b_u_col, b_i_col,
        W_intra_user.T, W_inter_user.T)
    return (new_usr, new_itm)
